# baseline (speedup 1.0000x reference)
import functools

import jax
import jax.numpy as jnp
import numpy as np
from jax import lax
from jax.experimental import pallas as pl
from jax.experimental.pallas import tpu as pltpu
from jax.experimental.pallas import tpu_sc as plsc

_DIM_LEN = 768
_GROUP = 4

_B = 2
_H = 224
_W = 224
_HH = 112
_NCH = _B * _DIM_LEN * 2
_NC = 2
_NS = 16
_NW = _NC * _NS
_CPW = _NCH // _NW


_PERM = np.array([
    183, 138, 166, 19, 76, 158, 118, 143, 54, 189, 149, 90, 30, 7, 96, 139,
    155, 131, 121, 115, 6, 35, 23, 58, 128, 16, 21, 156, 77, 154, 160, 94,
    116, 61, 38, 3, 185, 105, 132, 81, 26, 32, 64, 37, 56, 51, 2, 122,
    63, 133, 52, 20, 89, 95, 44, 47, 123, 79, 84, 144, 157, 135, 50, 140,
    78, 179, 72, 163, 191, 83, 42, 62, 152, 69, 53, 148, 172, 0, 145, 8,
    167, 169, 159, 109, 181, 22, 178, 13, 29, 99, 110, 34, 70, 175, 18, 103,
    141, 86, 142, 75, 187, 91, 111, 24, 113, 1, 65, 48, 5, 45, 165, 150,
    49, 173, 33, 74, 55, 182, 136, 60, 119, 57, 124, 27, 112, 129, 151, 10,
    134, 186, 93, 176, 161, 68, 146, 15, 73, 40, 67, 88, 102, 107, 66, 80,
    100, 120, 147, 71, 17, 59, 184, 98, 108, 114, 36, 125, 101, 180, 92, 171,
    153, 28, 46, 9, 104, 117, 4, 177, 170, 190, 130, 12, 168, 188, 87, 85,
    14, 174, 82, 31, 106, 127, 162, 126, 164, 97, 41, 137, 25, 43, 39, 11,
], dtype=np.int32)


def _src_channels() -> np.ndarray:
    chan = (_PERM[:, None] * _GROUP + np.arange(_GROUP)[None, :]).reshape(-1)
    return chan.astype(np.int32)


_SRC_CHANNELS = _src_channels()


def _permute_channels(x, src):
    mesh = plsc.VectorSubcoreMesh(core_axis_name="c", subcore_axis_name="s")

    @functools.partial(
        pl.kernel,
        mesh=mesh,
        out_type=jax.ShapeDtypeStruct((_B, _DIM_LEN, _H, _W), jnp.float32),
        compiler_params=pltpu.CompilerParams(needs_layout_passes=False),
        scratch_types=[
            pltpu.VMEM((_DIM_LEN,), jnp.int32),
            pltpu.VMEM((_HH, _W), jnp.float32),
            pltpu.VMEM((_HH, _W), jnp.float32),
            pltpu.VMEM((_HH, _W), jnp.float32),
            pltpu.VMEM((_HH, _W), jnp.float32),
            pltpu.SemaphoreType.DMA,
            pltpu.SemaphoreType.DMA,
        ],
    )
    def k(x_hbm, src_hbm, out_hbm, idx_v, buf0, buf1, buf2, buf3, gsem, wsem):
        wid = lax.axis_index("s") * _NC + lax.axis_index("c")
        base = wid * _CPW
        bufs = (buf0, buf1, buf2, buf3)
        pltpu.sync_copy(src_hbm, idx_v)
        lanes = lax.iota(jnp.int32, 16)

        def coords(c):
            q = base + c
            return q // (2 * _DIM_LEN), (q // 2) % _DIM_LEN, pl.multiple_of((q % 2) * _HH, _HH)

        def src_chan(c):
            oc = coords(c)[1]
            aligned = pl.multiple_of((oc // 16) * 16, 16)
            vec = idx_v[pl.ds(aligned, 16)]
            return jnp.max(jnp.where(lanes == oc - aligned, vec, 0))

        chans = [None] * _CPW

        def start_gather(c):
            b, _, hoff = coords(c)
            chans[c] = src_chan(c)
            pltpu.async_copy(
                x_hbm.at[b, chans[c], pl.ds(hoff, _HH)], bufs[c % 4], gsem
            )

        def wait_gather(c):
            b, _, hoff = coords(c)
            pltpu.make_async_copy(
                x_hbm.at[b, chans[c], pl.ds(hoff, _HH)], bufs[c % 4], gsem
            ).wait()

        def start_write(c):
            b, oc, hoff = coords(c)
            pltpu.async_copy(
                bufs[c % 4], out_hbm.at[b, oc, pl.ds(hoff, _HH)], wsem
            )

        def wait_write(c):
            b, oc, hoff = coords(c)
            pltpu.make_async_copy(
                bufs[c % 4], out_hbm.at[b, oc, pl.ds(hoff, _HH)], wsem
            ).wait()

        for c in range(_CPW):
            if c >= 4:
                wait_write(c - 4)
            start_gather(c)
            if c >= 2:
                wait_gather(c - 2)
                start_write(c - 2)
        for c in (_CPW - 2, _CPW - 1):
            wait_gather(c)
            start_write(c)
        for c in range(_CPW - 4, _CPW):
            wait_write(c)

    return k(x, src)


def kernel(x):
    src = jnp.asarray(_SRC_CHANNELS)
    return _permute_channels(x, src)


# speedup vs baseline: 1.0041x; 1.0041x over previous
import functools

import jax
import jax.numpy as jnp
import numpy as np
from jax import lax
from jax.experimental import pallas as pl
from jax.experimental.pallas import tpu as pltpu
from jax.experimental.pallas import tpu_sc as plsc

_DIM_LEN = 768
_GROUP = 4

_B = 2
_H = 224
_W = 224
_NCH = _B * _DIM_LEN
_NC = 2
_NS = 16
_NW = _NC * _NS
_CPW = _NCH // _NW


_PERM = np.array([
    183, 138, 166, 19, 76, 158, 118, 143, 54, 189, 149, 90, 30, 7, 96, 139,
    155, 131, 121, 115, 6, 35, 23, 58, 128, 16, 21, 156, 77, 154, 160, 94,
    116, 61, 38, 3, 185, 105, 132, 81, 26, 32, 64, 37, 56, 51, 2, 122,
    63, 133, 52, 20, 89, 95, 44, 47, 123, 79, 84, 144, 157, 135, 50, 140,
    78, 179, 72, 163, 191, 83, 42, 62, 152, 69, 53, 148, 172, 0, 145, 8,
    167, 169, 159, 109, 181, 22, 178, 13, 29, 99, 110, 34, 70, 175, 18, 103,
    141, 86, 142, 75, 187, 91, 111, 24, 113, 1, 65, 48, 5, 45, 165, 150,
    49, 173, 33, 74, 55, 182, 136, 60, 119, 57, 124, 27, 112, 129, 151, 10,
    134, 186, 93, 176, 161, 68, 146, 15, 73, 40, 67, 88, 102, 107, 66, 80,
    100, 120, 147, 71, 17, 59, 184, 98, 108, 114, 36, 125, 101, 180, 92, 171,
    153, 28, 46, 9, 104, 117, 4, 177, 170, 190, 130, 12, 168, 188, 87, 85,
    14, 174, 82, 31, 106, 127, 162, 126, 164, 97, 41, 137, 25, 43, 39, 11,
], dtype=np.int32)


def _src_channels() -> np.ndarray:
    chan = (_PERM[:, None] * _GROUP + np.arange(_GROUP)[None, :]).reshape(-1)
    return chan.astype(np.int32)


_SRC_CHANNELS = _src_channels()


def _permute_channels(x, src):
    mesh = plsc.VectorSubcoreMesh(core_axis_name="c", subcore_axis_name="s")

    @functools.partial(
        pl.kernel,
        mesh=mesh,
        out_type=jax.ShapeDtypeStruct((_B, _DIM_LEN, _H, _W), jnp.float32),
        compiler_params=pltpu.CompilerParams(needs_layout_passes=False),
        scratch_types=[
            pltpu.VMEM((_DIM_LEN,), jnp.int32),
            pltpu.VMEM((_H, _W), jnp.float32),
            pltpu.VMEM((_H, _W), jnp.float32),
            pltpu.SemaphoreType.DMA,
            pltpu.SemaphoreType.DMA,
        ],
    )
    def k(x_hbm, src_hbm, out_hbm, idx_v, buf0, buf1, gsem, wsem):
        wid = lax.axis_index("s") * _NC + lax.axis_index("c")
        base = wid * _CPW
        bufs = (buf0, buf1)
        pltpu.sync_copy(src_hbm, idx_v)
        lanes = lax.iota(jnp.int32, 16)

        def src_chan(c):
            oc = (base + c) % _DIM_LEN
            aligned = pl.multiple_of((oc // 16) * 16, 16)
            vec = idx_v[pl.ds(aligned, 16)]
            return jnp.max(jnp.where(lanes == oc - aligned, vec, 0))

        chans = [None] * _CPW

        def start_gather(c):
            b = (base + c) // _DIM_LEN
            chans[c] = src_chan(c)
            pltpu.async_copy(x_hbm.at[b, chans[c]], bufs[c % 2], gsem)

        def wait_gather(c):
            b = (base + c) // _DIM_LEN
            pltpu.make_async_copy(x_hbm.at[b, chans[c]], bufs[c % 2], gsem).wait()

        def start_write(c):
            b = (base + c) // _DIM_LEN
            pltpu.async_copy(
                bufs[c % 2], out_hbm.at[b, (base + c) % _DIM_LEN], wsem
            )

        def wait_write(c):
            b = (base + c) // _DIM_LEN
            pltpu.make_async_copy(
                bufs[c % 2], out_hbm.at[b, (base + c) % _DIM_LEN], wsem
            ).wait()

        for c in range(_CPW):
            if c >= 2:
                wait_write(c - 2)
            start_gather(c)
            if c >= 1:
                wait_gather(c - 1)
                start_write(c - 1)
        wait_gather(_CPW - 1)
        start_write(_CPW - 1)
        wait_write(_CPW - 2)
        wait_write(_CPW - 1)

    return k(x, src)


def kernel(x):
    src = jnp.asarray(_SRC_CHANNELS)
    return _permute_channels(x, src)
